# baseline (speedup 1.0000x reference)
import jax
import jax.numpy as jnp
from jax.experimental import pallas as pl
from jax.experimental.pallas import tpu as pltpu


_ROWS = 16384
_DIM = 128
_NCHUNK = 8
_CHUNK = _ROWS // _NCHUNK


def _dma_kernel(q_hbm, out_hbm, mask_ref, buf, in_sems, out_sems):
    mask_ref[...] = jnp.ones(mask_ref.shape, dtype=jnp.bool_)

    def in_copy(i):
        return pltpu.make_async_copy(
            q_hbm.at[pl.ds(i * _CHUNK, _CHUNK), :],
            buf.at[i],
            in_sems.at[i],
        )

    def out_copy(i):
        return pltpu.make_async_copy(
            buf.at[i],
            out_hbm.at[pl.ds(i * _CHUNK, _CHUNK), :],
            out_sems.at[i],
        )

    for i in range(_NCHUNK):
        in_copy(i).start()
    for i in range(_NCHUNK):
        in_copy(i).wait()
        out_copy(i).start()
    for i in range(_NCHUNK):
        out_copy(i).wait()


def kernel(query, relation_weight):
    out, mask = pl.pallas_call(
        _dma_kernel,
        in_specs=[pl.BlockSpec(memory_space=pl.ANY)],
        out_specs=[
            pl.BlockSpec(memory_space=pl.ANY),
            pl.BlockSpec((_DIM, _DIM), lambda: (0, 0)),
        ],
        out_shape=[
            jax.ShapeDtypeStruct((_ROWS, _DIM), jnp.float32),
            jax.ShapeDtypeStruct((_DIM, _DIM), jnp.bool_),
        ],
        scratch_shapes=[
            pltpu.VMEM((_NCHUNK, _CHUNK, _DIM), jnp.float32),
            pltpu.SemaphoreType.DMA((_NCHUNK,)),
            pltpu.SemaphoreType.DMA((_NCHUNK,)),
        ],
    )(query)
    return (out.reshape(_ROWS, 1, _DIM), mask.reshape(_ROWS, 1))


# speedup vs baseline: 1.0120x; 1.0120x over previous
import jax
import jax.numpy as jnp
from jax.experimental import pallas as pl
from jax.experimental.pallas import tpu as pltpu


_ROWS = 16384
_DIM = 128
_NCHUNK = 4
_CHUNK = _ROWS // _NCHUNK


def _dma_kernel(q_hbm, out_hbm, mask_ref, buf, in_sems, out_sems):
    mask_ref[...] = jnp.ones(mask_ref.shape, dtype=jnp.bool_)

    def in_copy(i):
        return pltpu.make_async_copy(
            q_hbm.at[pl.ds(i * _CHUNK, _CHUNK), :],
            buf.at[i],
            in_sems.at[i],
        )

    def out_copy(i):
        return pltpu.make_async_copy(
            buf.at[i],
            out_hbm.at[pl.ds(i * _CHUNK, _CHUNK), :],
            out_sems.at[i],
        )

    for i in range(_NCHUNK):
        in_copy(i).start()
    for i in range(_NCHUNK):
        in_copy(i).wait()
        out_copy(i).start()
    for i in range(_NCHUNK):
        out_copy(i).wait()


def kernel(query, relation_weight):
    out, mask = pl.pallas_call(
        _dma_kernel,
        in_specs=[pl.BlockSpec(memory_space=pl.ANY)],
        out_specs=[
            pl.BlockSpec(memory_space=pl.ANY),
            pl.BlockSpec((_DIM, _DIM), lambda: (0, 0)),
        ],
        out_shape=[
            jax.ShapeDtypeStruct((_ROWS, _DIM), jnp.float32),
            jax.ShapeDtypeStruct((_DIM, _DIM), jnp.bool_),
        ],
        scratch_shapes=[
            pltpu.VMEM((_NCHUNK, _CHUNK, _DIM), jnp.float32),
            pltpu.SemaphoreType.DMA((_NCHUNK,)),
            pltpu.SemaphoreType.DMA((_NCHUNK,)),
        ],
    )(query)
    return (out.reshape(_ROWS, 1, _DIM), mask.reshape(_ROWS, 1))


# speedup vs baseline: 1.0788x; 1.0659x over previous
import jax
import jax.numpy as jnp
from jax.experimental import pallas as pl


_ROWS = 16384
_DIM = 128
_BLOCK = 8192


def _copy_kernel(q_ref, out_ref, mask_ref):
    out_ref[...] = q_ref[...]

    @pl.when(pl.program_id(0) == 0)
    def _():
        mask_ref[...] = jnp.ones(mask_ref.shape, dtype=jnp.bool_)


def kernel(query, relation_weight):
    out, mask = pl.pallas_call(
        _copy_kernel,
        grid=(_ROWS // _BLOCK,),
        in_specs=[pl.BlockSpec((_BLOCK, _DIM), lambda i: (i, 0))],
        out_specs=[
            pl.BlockSpec((_BLOCK, _DIM), lambda i: (i, 0)),
            pl.BlockSpec((_DIM, _DIM), lambda i: (0, 0)),
        ],
        out_shape=[
            jax.ShapeDtypeStruct((_ROWS, _DIM), jnp.float32),
            jax.ShapeDtypeStruct((_DIM, _DIM), jnp.bool_),
        ],
    )(query)
    return (out.reshape(_ROWS, 1, _DIM), mask.reshape(_ROWS, 1))
